# Initial kernel scaffold; baseline (speedup 1.0000x reference)
import jax
import jax.numpy as jnp
from jax.experimental import pallas as pl


def kernel(x):
    raise NotImplementedError("write your pallas kernel here")


# speedup vs baseline: 15.6140x; 15.6140x over previous
import functools

import jax
import jax.numpy as jnp
from jax.experimental import pallas as pl
from jax.experimental.pallas import tpu as pltpu

K = 16
RB = 392


def _normalize_body(x_ref, xn_ref, xnt_ref):
    v = x_ref[0]
    sq = jnp.sum(v * v, axis=0, keepdims=True)
    n = jnp.sqrt(sq)
    xn = v / jnp.maximum(n, 1e-12)
    xn_ref[0] = xn
    xnt_ref[0] = xn.T


def _knn_body(xnt_ref, xn_ref, idx_ref, *, n_points):
    lhs = xnt_ref[0]
    rhs = xn_ref[0]
    sqi = jnp.sum(lhs * lhs, axis=1, keepdims=True)
    sqj = jnp.sum(rhs * rhs, axis=0, keepdims=True)
    g = jax.lax.dot_general(lhs, rhs, (((1,), (0,)), ((), ())),
                            preferred_element_type=jnp.float32)
    d = (sqi + (-2.0 * g)) + sqj
    work = -d
    it = jax.lax.broadcasted_iota(jnp.int32, (RB, n_points), 1)
    cols = []
    for _ in range(K):
        m = jnp.max(work, axis=1, keepdims=True)
        cand = jnp.where(work == m, it, n_points)
        idx = jnp.min(cand, axis=1, keepdims=True)
        cols.append(idx)
        work = jnp.where(it == idx, -jnp.inf, work)
    idx_ref[0] = jnp.concatenate(cols, axis=1)


def kernel(x):
    B, C, H, W = x.shape
    N = H * W
    xf = x.reshape(B, C, N)

    xn, xnt = pl.pallas_call(
        _normalize_body,
        grid=(B,),
        in_specs=[pl.BlockSpec((1, C, N), lambda b: (b, 0, 0))],
        out_specs=[
            pl.BlockSpec((1, C, N), lambda b: (b, 0, 0)),
            pl.BlockSpec((1, N, C), lambda b: (b, 0, 0)),
        ],
        out_shape=[
            jax.ShapeDtypeStruct((B, C, N), jnp.float32),
            jax.ShapeDtypeStruct((B, N, C), jnp.float32),
        ],
    )(xf)

    nn_idx = pl.pallas_call(
        functools.partial(_knn_body, n_points=N),
        grid=(B, N // RB),
        in_specs=[
            pl.BlockSpec((1, RB, C), lambda b, r: (b, r, 0)),
            pl.BlockSpec((1, C, N), lambda b, r: (b, 0, 0)),
        ],
        out_specs=pl.BlockSpec((1, RB, K), lambda b, r: (b, r, 0)),
        out_shape=jax.ShapeDtypeStruct((B, N, K), jnp.int32),
    )(xnt, xn)

    center_idx = jnp.broadcast_to(
        jnp.arange(N, dtype=jnp.int32)[None, :, None], (B, N, K))
    return jnp.stack((nn_idx, center_idx), axis=0)
